# Initial kernel scaffold; baseline (speedup 1.0000x reference)
import jax
import jax.numpy as jnp
from jax.experimental import pallas as pl


def kernel(x, edge_index, W1, b1, W2, b2):
    raise NotImplementedError("write your pallas kernel here")


# speedup vs baseline: 13.2978x; 13.2978x over previous
import functools

import jax
import jax.numpy as jnp
from jax import lax
from jax.experimental import pallas as pl
from jax.experimental.pallas import tpu as pltpu
from jax.experimental.pallas import tpu_sc as plsc

N = 10000
E = 160000
NC = 2
NS = 16
NTM = 624
REM = N - NTM * NS
ET = E // NS
ED = E // (NC * NS)
CD = 1000

R = 1000

_MESH = plsc.VectorSubcoreMesh(
    core_axis_name="c", subcore_axis_name="s", num_cores=NC, num_subcores=NS)
_SC_PARAMS = pltpu.CompilerParams(use_tc_tiling_on_sc=False)


@functools.partial(
    pl.kernel,
    out_type=jax.ShapeDtypeStruct((NC * N, 16), jnp.float32),
    mesh=_MESH,
    compiler_params=_SC_PARAMS,
    scratch_types=[
        pltpu.VMEM_SHARED((N, 16), jnp.float32),
        pltpu.VMEM((NTM, 16), jnp.float32),
        pltpu.VMEM((CD, 16), jnp.float32),
        pltpu.VMEM((CD,), jnp.int32),
        pltpu.SemaphoreType.DMA,
    ],
)
def _deg_kernel(col_hbm, zeros_hbm, ones_hbm, out_hbm, acc, zed_v, ones_v,
                icol_v, sem):
    c = lax.axis_index("c")
    s = lax.axis_index("s")
    pltpu.sync_copy(zeros_hbm.at[pl.ds(0, NTM), :], zed_v)
    pltpu.sync_copy(ones_hbm.at[pl.ds(0, CD), :], ones_v)
    pltpu.sync_copy(zed_v, acc.at[pl.ds(s * NTM, NTM), :])

    @pl.when(s == 0)
    def _():
        pltpu.sync_copy(zed_v.at[pl.ds(0, REM), :],
                        acc.at[pl.ds(NTM * NS, REM), :])

    plsc.subcore_barrier()
    wid = s * NC + c
    e_base = wid * ED

    def body(i, _):
        pltpu.sync_copy(col_hbm.at[pl.ds(e_base + i * CD, CD)], icol_v)
        pltpu.sync_copy(ones_v, acc.at[icol_v], add=True)
        return 0

    lax.fori_loop(0, ED // CD, body, 0)
    plsc.subcore_barrier()
    pltpu.sync_copy(acc.at[pl.ds(s * NTM, NTM), :],
                    out_hbm.at[pl.ds(c * N + s * NTM, NTM), :])

    @pl.when(s == 0)
    def _():
        pltpu.sync_copy(acc.at[pl.ds(NTM * NS, REM), :],
                        out_hbm.at[pl.ds(c * N + NTM * NS, REM), :])


def _make_agg(dh, CA):
    @functools.partial(
        pl.kernel,
        out_type=jax.ShapeDtypeStruct((NC * N, dh), jnp.float32),
        mesh=_MESH,
        compiler_params=_SC_PARAMS,
        scratch_types=[
            pltpu.VMEM_SHARED((N, dh), jnp.float32),
            pltpu.VMEM((CA, dh), jnp.float32),
            pltpu.VMEM((CA,), jnp.int32),
            pltpu.VMEM((CA,), jnp.int32),
            pltpu.SemaphoreType.DMA,
        ],
    )
    def agg(g_hbm, row_hbm, col_hbm, out_hbm, acc, rows_v, irow_v, icol_v,
            sem):
        c = lax.axis_index("c")
        s = lax.axis_index("s")
        pltpu.sync_copy(g_hbm.at[pl.ds(c * N + s * NTM, NTM), :],
                        acc.at[pl.ds(s * NTM, NTM), :])

        @pl.when(s == 0)
        def _():
            pltpu.sync_copy(g_hbm.at[pl.ds(c * N + NTM * NS, REM), :],
                            acc.at[pl.ds(NTM * NS, REM), :])

        plsc.subcore_barrier()
        e_base = c * E + s * ET

        def body(i, _):
            base = e_base + i * CA
            pltpu.sync_copy(row_hbm.at[pl.ds(base, CA)], irow_v)
            pltpu.sync_copy(col_hbm.at[pl.ds(base - c * E, CA)], icol_v)
            pltpu.async_copy(g_hbm.at[irow_v], rows_v, sem).wait()
            pltpu.sync_copy(rows_v, acc.at[icol_v], add=True)
            return 0

        lax.fori_loop(0, ET // CA, body, 0)
        plsc.subcore_barrier()
        pltpu.sync_copy(acc.at[pl.ds(s * NTM, NTM), :],
                        out_hbm.at[pl.ds(c * N + s * NTM, NTM), :])

        @pl.when(s == 0)
        def _():
            pltpu.sync_copy(acc.at[pl.ds(NTM * NS, REM), :],
                            out_hbm.at[pl.ds(c * N + NTM * NS, REM), :])

    return agg


_agg64 = _make_agg(64, 400)
_agg128 = _make_agg(128, 200)


def _dis_from(deg_ref):
    return lax.rsqrt(deg_ref[0, :, 0] + deg_ref[1, :, 0] + 1.0)


def _tc_a_body(x_ref, w_ref, deg_ref, out_ref):
    h = jnp.dot(x_ref[...], w_ref[...], preferred_element_type=jnp.float32)
    g = h * _dis_from(deg_ref)[:, None]
    out_ref[0] = g[:, :64]
    out_ref[1] = g[:, 64:]


def _tc_b_body(s_ref, deg_ref, b_ref, w_ref, out_ref):
    dis = _dis_from(deg_ref)
    sfull = jnp.concatenate([s_ref[0], s_ref[1]], axis=1)
    x2 = jnp.maximum(sfull * dis[:, None] + b_ref[0], 0.0)
    h = jnp.dot(x2, w_ref[...], preferred_element_type=jnp.float32)
    g = h * dis[:, None]
    out_ref[0] = g[:, :128]
    out_ref[1] = g[:, 128:]


def _tc_c_body(s_ref, deg_ref, b_ref, out_ref):
    dis = _dis_from(deg_ref)
    sfull = jnp.concatenate([s_ref[0], s_ref[1]], axis=1)
    out_ref[...] = sfull * dis[:, None] + b_ref[0]


_tc_a = pl.pallas_call(
    _tc_a_body,
    grid=(N // R,),
    in_specs=[
        pl.BlockSpec((R, 256), lambda i: (i, 0)),
        pl.BlockSpec((256, 128), lambda i: (0, 0)),
        pl.BlockSpec((2, R, 16), lambda i: (0, i, 0)),
    ],
    out_specs=pl.BlockSpec((2, R, 64), lambda i: (0, i, 0)),
    out_shape=jax.ShapeDtypeStruct((2, N, 64), jnp.float32),
)

_tc_b = pl.pallas_call(
    _tc_b_body,
    grid=(N // R,),
    in_specs=[
        pl.BlockSpec((2, R, 64), lambda i: (0, i, 0)),
        pl.BlockSpec((2, R, 16), lambda i: (0, i, 0)),
        pl.BlockSpec((1, 128), lambda i: (0, 0)),
        pl.BlockSpec((128, 256), lambda i: (0, 0)),
    ],
    out_specs=pl.BlockSpec((2, R, 128), lambda i: (0, i, 0)),
    out_shape=jax.ShapeDtypeStruct((2, N, 128), jnp.float32),
)

_tc_c = pl.pallas_call(
    _tc_c_body,
    grid=(N // R,),
    in_specs=[
        pl.BlockSpec((2, R, 128), lambda i: (0, i, 0)),
        pl.BlockSpec((2, R, 16), lambda i: (0, i, 0)),
        pl.BlockSpec((1, 256), lambda i: (0, 0)),
    ],
    out_specs=pl.BlockSpec((R, 256), lambda i: (i, 0)),
    out_shape=jax.ShapeDtypeStruct((N, 256), jnp.float32),
)


@jax.jit
def kernel(x, edge_index, W1, b1, W2, b2):
    row = edge_index[0]
    col = edge_index[1]
    rowcat = jnp.concatenate([row, row + N])
    zeros16 = jnp.zeros((NTM, 16), jnp.float32)
    ones16 = jnp.ones((CD, 16), jnp.float32)

    deg2 = _deg_kernel(col, zeros16, ones16).reshape(2, N, 16)

    g1 = _tc_a(x, W1, deg2).reshape(2 * N, 64)
    s1 = _agg64(g1, rowcat, col).reshape(2, N, 64)

    g2 = _tc_b(s1, deg2, b1.reshape(1, 128), W2).reshape(2 * N, 128)
    s2 = _agg128(g2, rowcat, col).reshape(2, N, 128)

    return _tc_c(s2, deg2, b2.reshape(1, 256))


# speedup vs baseline: 18.6192x; 1.4002x over previous
import functools

import jax
import jax.numpy as jnp
from jax import lax
from jax.experimental import pallas as pl
from jax.experimental.pallas import tpu as pltpu
from jax.experimental.pallas import tpu_sc as plsc

N = 10000
E = 160000
NC = 2
NS = 16
NTM = 624
REM = N - NTM * NS
ET = E // NS
ED = E // (NC * NS)
CD = 1000

R = 1000

_MESH = plsc.VectorSubcoreMesh(
    core_axis_name="c", subcore_axis_name="s", num_cores=NC, num_subcores=NS)
_SC_PARAMS = pltpu.CompilerParams(use_tc_tiling_on_sc=False)


@functools.partial(
    pl.kernel,
    out_type=jax.ShapeDtypeStruct((NC * N, 16), jnp.float32),
    mesh=_MESH,
    compiler_params=_SC_PARAMS,
    scratch_types=[
        pltpu.VMEM_SHARED((N, 16), jnp.float32),
        pltpu.VMEM((NTM, 16), jnp.float32),
        pltpu.VMEM((CD, 16), jnp.float32),
        pltpu.VMEM((CD,), jnp.int32),
        pltpu.SemaphoreType.DMA,
    ],
)
def _deg_kernel(col_hbm, zeros_hbm, ones_hbm, out_hbm, acc, zed_v, ones_v,
                icol_v, sem):
    c = lax.axis_index("c")
    s = lax.axis_index("s")
    pltpu.sync_copy(zeros_hbm.at[pl.ds(0, NTM), :], zed_v)
    pltpu.sync_copy(ones_hbm.at[pl.ds(0, CD), :], ones_v)
    pltpu.sync_copy(zed_v, acc.at[pl.ds(s * NTM, NTM), :])

    @pl.when(s == 0)
    def _():
        pltpu.sync_copy(zed_v.at[pl.ds(0, REM), :],
                        acc.at[pl.ds(NTM * NS, REM), :])

    plsc.subcore_barrier()
    wid = s * NC + c
    e_base = wid * ED

    def body(i, _):
        pltpu.sync_copy(col_hbm.at[pl.ds(e_base + i * CD, CD)], icol_v)
        pltpu.sync_copy(ones_v, acc.at[icol_v], add=True)
        return 0

    lax.fori_loop(0, ED // CD, body, 0)
    plsc.subcore_barrier()
    pltpu.sync_copy(acc.at[pl.ds(s * NTM, NTM), :],
                    out_hbm.at[pl.ds(c * N + s * NTM, NTM), :])

    @pl.when(s == 0)
    def _():
        pltpu.sync_copy(acc.at[pl.ds(NTM * NS, REM), :],
                        out_hbm.at[pl.ds(c * N + NTM * NS, REM), :])


CA = 100
NCH = ET // CA
NP = NCH // 2


def _make_agg(dh):
    @functools.partial(
        pl.kernel,
        out_type=jax.ShapeDtypeStruct((NC * N, dh), jnp.float32),
        mesh=_MESH,
        compiler_params=_SC_PARAMS,
        scratch_types=[
            pltpu.VMEM_SHARED((N, dh), jnp.float32),
            pltpu.VMEM((NCH, CA), jnp.int32),
            pltpu.VMEM((NCH, CA), jnp.int32),
            pltpu.VMEM((CA, dh), jnp.float32),
            pltpu.VMEM((CA, dh), jnp.float32),
            pltpu.SemaphoreType.DMA,
            pltpu.SemaphoreType.DMA,
        ],
    )
    def agg(g_hbm, row2d_hbm, col2d_hbm, out_hbm, acc, irow2d, icol2d,
            rows0, rows1, semg0, semg1):
        c = lax.axis_index("c")
        s = lax.axis_index("s")
        pltpu.sync_copy(g_hbm.at[pl.ds(c * N + s * NTM, NTM), :],
                        acc.at[pl.ds(s * NTM, NTM), :])

        @pl.when(s == 0)
        def _():
            pltpu.sync_copy(g_hbm.at[pl.ds(c * N + NTM * NS, REM), :],
                            acc.at[pl.ds(NTM * NS, REM), :])

        pltpu.sync_copy(
            row2d_hbm.at[pl.ds((c * E + s * ET) // CA, NCH), :], irow2d)
        pltpu.sync_copy(col2d_hbm.at[pl.ds(s * ET // CA, NCH), :], icol2d)
        plsc.subcore_barrier()

        pltpu.async_copy(g_hbm.at[irow2d.at[0]], rows0, semg0)

        def body(k, _):
            a = 2 * k
            pltpu.async_copy(g_hbm.at[irow2d.at[a + 1]], rows1, semg1)
            pltpu.make_async_copy(g_hbm.at[irow2d.at[a]], rows0, semg0).wait()
            pltpu.sync_copy(rows0, acc.at[icol2d.at[a]], add=True)

            @pl.when(k < NP - 1)
            def _():
                pltpu.async_copy(g_hbm.at[irow2d.at[a + 2]], rows0, semg0)

            pltpu.make_async_copy(
                g_hbm.at[irow2d.at[a + 1]], rows1, semg1).wait()
            pltpu.sync_copy(rows1, acc.at[icol2d.at[a + 1]], add=True)
            return 0

        lax.fori_loop(0, NP, body, 0)
        plsc.subcore_barrier()
        pltpu.sync_copy(acc.at[pl.ds(s * NTM, NTM), :],
                        out_hbm.at[pl.ds(c * N + s * NTM, NTM), :])

        @pl.when(s == 0)
        def _():
            pltpu.sync_copy(acc.at[pl.ds(NTM * NS, REM), :],
                            out_hbm.at[pl.ds(c * N + NTM * NS, REM), :])

    return agg


_agg64 = _make_agg(64)
_agg128 = _make_agg(128)


def _dis_from(deg_ref):
    return lax.rsqrt(deg_ref[0, :, 0] + deg_ref[1, :, 0] + 1.0)


def _tc_a_body(x_ref, w_ref, deg_ref, out_ref):
    h = jnp.dot(x_ref[...], w_ref[...], preferred_element_type=jnp.float32)
    g = h * _dis_from(deg_ref)[:, None]
    out_ref[0] = g[:, :64]
    out_ref[1] = g[:, 64:]


def _tc_b_body(s_ref, deg_ref, b_ref, w_ref, out_ref):
    dis = _dis_from(deg_ref)
    sfull = jnp.concatenate([s_ref[0], s_ref[1]], axis=1)
    x2 = jnp.maximum(sfull * dis[:, None] + b_ref[0], 0.0)
    h = jnp.dot(x2, w_ref[...], preferred_element_type=jnp.float32)
    g = h * dis[:, None]
    out_ref[0] = g[:, :128]
    out_ref[1] = g[:, 128:]


def _tc_c_body(s_ref, deg_ref, b_ref, out_ref):
    dis = _dis_from(deg_ref)
    sfull = jnp.concatenate([s_ref[0], s_ref[1]], axis=1)
    out_ref[...] = sfull * dis[:, None] + b_ref[0]


_tc_a = pl.pallas_call(
    _tc_a_body,
    grid=(N // R,),
    in_specs=[
        pl.BlockSpec((R, 256), lambda i: (i, 0)),
        pl.BlockSpec((256, 128), lambda i: (0, 0)),
        pl.BlockSpec((2, R, 16), lambda i: (0, i, 0)),
    ],
    out_specs=pl.BlockSpec((2, R, 64), lambda i: (0, i, 0)),
    out_shape=jax.ShapeDtypeStruct((2, N, 64), jnp.float32),
)

_tc_b = pl.pallas_call(
    _tc_b_body,
    grid=(N // R,),
    in_specs=[
        pl.BlockSpec((2, R, 64), lambda i: (0, i, 0)),
        pl.BlockSpec((2, R, 16), lambda i: (0, i, 0)),
        pl.BlockSpec((1, 128), lambda i: (0, 0)),
        pl.BlockSpec((128, 256), lambda i: (0, 0)),
    ],
    out_specs=pl.BlockSpec((2, R, 128), lambda i: (0, i, 0)),
    out_shape=jax.ShapeDtypeStruct((2, N, 128), jnp.float32),
)

_tc_c = pl.pallas_call(
    _tc_c_body,
    grid=(N // R,),
    in_specs=[
        pl.BlockSpec((2, R, 128), lambda i: (0, i, 0)),
        pl.BlockSpec((2, R, 16), lambda i: (0, i, 0)),
        pl.BlockSpec((1, 256), lambda i: (0, 0)),
    ],
    out_specs=pl.BlockSpec((R, 256), lambda i: (i, 0)),
    out_shape=jax.ShapeDtypeStruct((N, 256), jnp.float32),
)


@jax.jit
def kernel(x, edge_index, W1, b1, W2, b2):
    row = edge_index[0]
    col = edge_index[1]
    row2d = jnp.concatenate([row, row + N]).reshape(2 * E // CA, CA)
    col2d = col.reshape(E // CA, CA)
    zeros16 = jnp.zeros((NTM, 16), jnp.float32)
    ones16 = jnp.ones((CD, 16), jnp.float32)

    deg2 = _deg_kernel(col, zeros16, ones16).reshape(2, N, 16)

    g1 = _tc_a(x, W1, deg2).reshape(2 * N, 64)
    s1 = _agg64(g1, row2d, col2d).reshape(2, N, 64)

    g2 = _tc_b(s1, deg2, b1.reshape(1, 128), W2).reshape(2 * N, 128)
    s2 = _agg128(g2, row2d, col2d).reshape(2, N, 128)

    return _tc_c(s2, deg2, b2.reshape(1, 256))


# speedup vs baseline: 21.4509x; 1.1521x over previous
import functools

import jax
import jax.numpy as jnp
from jax import lax
from jax.experimental import pallas as pl
from jax.experimental.pallas import tpu as pltpu
from jax.experimental.pallas import tpu_sc as plsc

N = 10000
E = 160000
NC = 2
NS = 16
NTM = 624
REM = N - NTM * NS
ET = E // NS
ED = E // (NC * NS)
CD = 1000

R = 1000

_MESH = plsc.VectorSubcoreMesh(
    core_axis_name="c", subcore_axis_name="s", num_cores=NC, num_subcores=NS)
_SC_PARAMS = pltpu.CompilerParams(use_tc_tiling_on_sc=False)


@functools.partial(
    pl.kernel,
    out_type=jax.ShapeDtypeStruct((NC * N, 16), jnp.float32),
    mesh=_MESH,
    compiler_params=_SC_PARAMS,
    scratch_types=[
        pltpu.VMEM_SHARED((N, 16), jnp.float32),
        pltpu.VMEM((NTM, 16), jnp.float32),
        pltpu.VMEM((CD, 16), jnp.float32),
        pltpu.VMEM((CD,), jnp.int32),
        pltpu.SemaphoreType.DMA,
    ],
)
def _deg_kernel(col_hbm, zeros_hbm, ones_hbm, out_hbm, acc, zed_v, ones_v,
                icol_v, sem):
    c = lax.axis_index("c")
    s = lax.axis_index("s")
    pltpu.sync_copy(zeros_hbm.at[pl.ds(0, NTM), :], zed_v)
    pltpu.sync_copy(ones_hbm.at[pl.ds(0, CD), :], ones_v)
    pltpu.sync_copy(zed_v, acc.at[pl.ds(s * NTM, NTM), :])

    @pl.when(s == 0)
    def _():
        pltpu.sync_copy(zed_v.at[pl.ds(0, REM), :],
                        acc.at[pl.ds(NTM * NS, REM), :])

    plsc.subcore_barrier()
    wid = s * NC + c
    e_base = wid * ED

    def body(i, _):
        pltpu.sync_copy(col_hbm.at[pl.ds(e_base + i * CD, CD)], icol_v)
        pltpu.sync_copy(ones_v, acc.at[icol_v], add=True)
        return 0

    lax.fori_loop(0, ED // CD, body, 0)
    plsc.subcore_barrier()
    pltpu.sync_copy(acc.at[pl.ds(s * NTM, NTM), :],
                    out_hbm.at[pl.ds(c * N + s * NTM, NTM), :])

    @pl.when(s == 0)
    def _():
        pltpu.sync_copy(acc.at[pl.ds(NTM * NS, REM), :],
                        out_hbm.at[pl.ds(c * N + NTM * NS, REM), :])


RING = 4


def _make_agg(dh, ca):
    nch = ET // ca
    nq = nch // RING

    def idxrow(idx2d, chunk):
        return idx2d.at[chunk]

    @functools.partial(
        pl.kernel,
        out_type=jax.ShapeDtypeStruct((NC * N, dh), jnp.float32),
        mesh=_MESH,
        compiler_params=_SC_PARAMS,
        scratch_types=[
            pltpu.VMEM_SHARED((N, dh), jnp.float32),
            pltpu.VMEM((nch, ca), jnp.int32),
            pltpu.VMEM((nch, ca), jnp.int32),
            [pltpu.VMEM((ca, dh), jnp.float32)] * RING,
            [pltpu.SemaphoreType.DMA] * RING,
            [pltpu.SemaphoreType.DMA] * RING,
        ],
    )
    def agg(g_hbm, row2d_hbm, col2d_hbm, out_hbm, acc, irow2d, icol2d,
            bufs, semg, sems):
        c = lax.axis_index("c")
        s = lax.axis_index("s")
        pltpu.sync_copy(g_hbm.at[pl.ds(c * N + s * NTM, NTM), :],
                        acc.at[pl.ds(s * NTM, NTM), :])

        @pl.when(s == 0)
        def _():
            pltpu.sync_copy(g_hbm.at[pl.ds(c * N + NTM * NS, REM), :],
                            acc.at[pl.ds(NTM * NS, REM), :])

        pltpu.sync_copy(
            row2d_hbm.at[pl.ds((c * E + s * ET) // ca, nch), :], irow2d)
        pltpu.sync_copy(col2d_hbm.at[pl.ds(s * ET // ca, nch), :], icol2d)
        plsc.subcore_barrier()

        def gat(chunk, b):
            return pltpu.make_async_copy(
                g_hbm.at[idxrow(irow2d, chunk)], bufs[b], semg[b])

        def sca(chunk, b):
            return pltpu.make_async_copy(
                bufs[b], acc.at[idxrow(icol2d, chunk)], sems[b])

        for b in range(RING):
            gat(b, b).start()

        def body(k, _):
            base = k * RING
            for b in range(RING):
                chunk = base + b
                gat(chunk, b).wait()
                pltpu.async_copy(
                    bufs[b], acc.at[idxrow(icol2d, chunk)], sems[b],
                    add=True)

                @pl.when(k < nq - 1)
                def _():
                    sca(chunk, b).wait()
                    gat(chunk + RING, b).start()

            return 0

        lax.fori_loop(0, nq, body, 0)
        for b in range(RING):
            sca(nch - RING + b, b).wait()
        plsc.subcore_barrier()
        pltpu.sync_copy(acc.at[pl.ds(s * NTM, NTM), :],
                        out_hbm.at[pl.ds(c * N + s * NTM, NTM), :])

        @pl.when(s == 0)
        def _():
            pltpu.sync_copy(acc.at[pl.ds(NTM * NS, REM), :],
                            out_hbm.at[pl.ds(c * N + NTM * NS, REM), :])

    return agg


_agg64 = _make_agg(64, 100)
_agg128 = _make_agg(128, 50)


def _dis_from(deg_ref):
    return lax.rsqrt(deg_ref[0, :, 0] + deg_ref[1, :, 0] + 1.0)


def _tc_a_body(x_ref, w_ref, deg_ref, out_ref):
    h = jnp.dot(x_ref[...], w_ref[...], preferred_element_type=jnp.float32)
    g = h * _dis_from(deg_ref)[:, None]
    out_ref[0] = g[:, :64]
    out_ref[1] = g[:, 64:]


def _tc_b_body(s_ref, deg_ref, b_ref, w_ref, out_ref):
    dis = _dis_from(deg_ref)
    sfull = jnp.concatenate([s_ref[0], s_ref[1]], axis=1)
    x2 = jnp.maximum(sfull * dis[:, None] + b_ref[0], 0.0)
    h = jnp.dot(x2, w_ref[...], preferred_element_type=jnp.float32)
    g = h * dis[:, None]
    out_ref[0] = g[:, :128]
    out_ref[1] = g[:, 128:]


def _tc_c_body(s_ref, deg_ref, b_ref, out_ref):
    dis = _dis_from(deg_ref)
    sfull = jnp.concatenate([s_ref[0], s_ref[1]], axis=1)
    out_ref[...] = sfull * dis[:, None] + b_ref[0]


_tc_a = pl.pallas_call(
    _tc_a_body,
    grid=(N // R,),
    in_specs=[
        pl.BlockSpec((R, 256), lambda i: (i, 0)),
        pl.BlockSpec((256, 128), lambda i: (0, 0)),
        pl.BlockSpec((2, R, 16), lambda i: (0, i, 0)),
    ],
    out_specs=pl.BlockSpec((2, R, 64), lambda i: (0, i, 0)),
    out_shape=jax.ShapeDtypeStruct((2, N, 64), jnp.float32),
)

_tc_b = pl.pallas_call(
    _tc_b_body,
    grid=(N // R,),
    in_specs=[
        pl.BlockSpec((2, R, 64), lambda i: (0, i, 0)),
        pl.BlockSpec((2, R, 16), lambda i: (0, i, 0)),
        pl.BlockSpec((1, 128), lambda i: (0, 0)),
        pl.BlockSpec((128, 256), lambda i: (0, 0)),
    ],
    out_specs=pl.BlockSpec((2, R, 128), lambda i: (0, i, 0)),
    out_shape=jax.ShapeDtypeStruct((2, N, 128), jnp.float32),
)

_tc_c = pl.pallas_call(
    _tc_c_body,
    grid=(N // R,),
    in_specs=[
        pl.BlockSpec((2, R, 128), lambda i: (0, i, 0)),
        pl.BlockSpec((2, R, 16), lambda i: (0, i, 0)),
        pl.BlockSpec((1, 256), lambda i: (0, 0)),
    ],
    out_specs=pl.BlockSpec((R, 256), lambda i: (i, 0)),
    out_shape=jax.ShapeDtypeStruct((N, 256), jnp.float32),
)


@jax.jit
def kernel(x, edge_index, W1, b1, W2, b2):
    row = edge_index[0]
    col = edge_index[1]
    rowcat = jnp.concatenate([row, row + N])
    zeros16 = jnp.zeros((NTM, 16), jnp.float32)
    ones16 = jnp.ones((CD, 16), jnp.float32)

    deg2 = _deg_kernel(col, zeros16, ones16).reshape(2, N, 16)

    g1 = _tc_a(x, W1, deg2).reshape(2 * N, 64)
    s1 = _agg64(g1, rowcat.reshape(-1, 100), col.reshape(-1, 100))
    s1 = s1.reshape(2, N, 64)

    g2 = _tc_b(s1, deg2, b1.reshape(1, 128), W2).reshape(2 * N, 128)
    s2 = _agg128(g2, rowcat.reshape(-1, 50), col.reshape(-1, 50))
    s2 = s2.reshape(2, N, 128)

    return _tc_c(s2, deg2, b2.reshape(1, 256))


# speedup vs baseline: 22.2869x; 1.0390x over previous
import functools

import jax
import jax.numpy as jnp
from jax import lax
from jax.experimental import pallas as pl
from jax.experimental.pallas import tpu as pltpu
from jax.experimental.pallas import tpu_sc as plsc

N = 10000
E = 160000
NC = 2
NS = 16
NTM = 624
REM = N - NTM * NS
ET = E // NS
ED = E // (NC * NS)
CD = 1000

R = 1000

_MESH = plsc.VectorSubcoreMesh(
    core_axis_name="c", subcore_axis_name="s", num_cores=NC, num_subcores=NS)
_SC_PARAMS = pltpu.CompilerParams(use_tc_tiling_on_sc=False)


@functools.partial(
    pl.kernel,
    out_type=jax.ShapeDtypeStruct((NC * N, 16), jnp.float32),
    mesh=_MESH,
    compiler_params=_SC_PARAMS,
    scratch_types=[
        pltpu.VMEM_SHARED((N, 16), jnp.float32),
        pltpu.VMEM((NTM, 16), jnp.float32),
        pltpu.VMEM((CD, 16), jnp.float32),
        pltpu.VMEM((CD,), jnp.int32),
        pltpu.SemaphoreType.DMA,
    ],
)
def _deg_kernel(col_hbm, zeros_hbm, ones_hbm, out_hbm, acc, zed_v, ones_v,
                icol_v, sem):
    c = lax.axis_index("c")
    s = lax.axis_index("s")
    pltpu.sync_copy(zeros_hbm.at[pl.ds(0, NTM), :], zed_v)
    pltpu.sync_copy(ones_hbm.at[pl.ds(0, CD), :], ones_v)
    pltpu.sync_copy(zed_v, acc.at[pl.ds(s * NTM, NTM), :])

    @pl.when(s == 0)
    def _():
        pltpu.sync_copy(zed_v.at[pl.ds(0, REM), :],
                        acc.at[pl.ds(NTM * NS, REM), :])

    plsc.subcore_barrier()
    wid = s * NC + c
    e_base = wid * ED

    def body(i, _):
        pltpu.sync_copy(col_hbm.at[pl.ds(e_base + i * CD, CD)], icol_v)
        pltpu.sync_copy(ones_v, acc.at[icol_v], add=True)
        return 0

    lax.fori_loop(0, ED // CD, body, 0)
    plsc.subcore_barrier()
    pltpu.sync_copy(acc.at[pl.ds(s * NTM, NTM), :],
                    out_hbm.at[pl.ds(c * N + s * NTM, NTM), :])

    @pl.when(s == 0)
    def _():
        pltpu.sync_copy(acc.at[pl.ds(NTM * NS, REM), :],
                        out_hbm.at[pl.ds(c * N + NTM * NS, REM), :])


RING = 4


def _make_agg(dh, ca, esplit):
    et = E // NC // NS if esplit else ET
    nch = et // ca
    nq = nch // RING

    def idxrow(idx2d, chunk):
        return idx2d.at[chunk]

    @functools.partial(
        pl.kernel,
        out_type=jax.ShapeDtypeStruct((NC * N, dh), jnp.float32),
        mesh=_MESH,
        compiler_params=_SC_PARAMS,
        scratch_types=[
            pltpu.VMEM_SHARED((N, dh), jnp.float32),
            pltpu.VMEM((nch, ca), jnp.int32),
            pltpu.VMEM((nch, ca), jnp.int32),
            [pltpu.VMEM((ca, dh), jnp.float32)] * RING,
            [pltpu.SemaphoreType.DMA] * RING,
            [pltpu.SemaphoreType.DMA] * RING,
        ],
    )
    def agg(g_hbm, row2d_hbm, col2d_hbm, out_hbm, acc, irow2d, icol2d,
            bufs, semg, sems):
        c = lax.axis_index("c")
        s = lax.axis_index("s")
        ginit = 0 if esplit else c * N
        pltpu.sync_copy(g_hbm.at[pl.ds(ginit + s * NTM, NTM), :],
                        acc.at[pl.ds(s * NTM, NTM), :])

        @pl.when(s == 0)
        def _():
            pltpu.sync_copy(g_hbm.at[pl.ds(ginit + NTM * NS, REM), :],
                            acc.at[pl.ds(NTM * NS, REM), :])

        if esplit:
            rbase = (c * (E // NC) + s * et) // ca
            cbase = rbase
        else:
            rbase = (c * E + s * et) // ca
            cbase = s * et // ca
        pltpu.sync_copy(row2d_hbm.at[pl.ds(rbase, nch), :], irow2d)
        pltpu.sync_copy(col2d_hbm.at[pl.ds(cbase, nch), :], icol2d)
        plsc.subcore_barrier()

        def gat(chunk, b):
            return pltpu.make_async_copy(
                g_hbm.at[idxrow(irow2d, chunk)], bufs[b], semg[b])

        def sca(chunk, b):
            return pltpu.make_async_copy(
                bufs[b], acc.at[idxrow(icol2d, chunk)], sems[b])

        for b in range(RING):
            gat(b, b).start()

        def body(k, _):
            base = k * RING
            for b in range(RING):
                chunk = base + b
                gat(chunk, b).wait()
                pltpu.async_copy(
                    bufs[b], acc.at[idxrow(icol2d, chunk)], sems[b],
                    add=True)

                @pl.when(k < nq - 1)
                def _():
                    sca(chunk, b).wait()
                    gat(chunk + RING, b).start()

            return 0

        lax.fori_loop(0, nq, body, 0)
        for b in range(RING):
            sca(nch - RING + b, b).wait()
        plsc.subcore_barrier()
        pltpu.sync_copy(acc.at[pl.ds(s * NTM, NTM), :],
                        out_hbm.at[pl.ds(c * N + s * NTM, NTM), :])

        @pl.when(s == 0)
        def _():
            pltpu.sync_copy(acc.at[pl.ds(NTM * NS, REM), :],
                            out_hbm.at[pl.ds(c * N + NTM * NS, REM), :])

    return agg


_agg_l1 = _make_agg(128, 50, esplit=True)
_agg_l2 = _make_agg(128, 50, esplit=False)


def _dis_from(deg_ref):
    return lax.rsqrt(deg_ref[0, :, 0] + deg_ref[1, :, 0] + 1.0)


def _tc_a0_body(x_ref, w_ref, out_ref):
    out_ref[...] = jnp.dot(x_ref[...], w_ref[...],
                           preferred_element_type=jnp.float32)


def _tc_a1_body(h_ref, deg_ref, out_ref):
    out_ref[...] = h_ref[...] * _dis_from(deg_ref)[:, None]


def _tc_b_body(s_ref, g1_ref, deg_ref, b_ref, w_ref, out_ref):
    dis = _dis_from(deg_ref)
    sfull = s_ref[0] + s_ref[1] - g1_ref[...]
    x2 = jnp.maximum(sfull * dis[:, None] + b_ref[0], 0.0)
    h = jnp.dot(x2, w_ref[...], preferred_element_type=jnp.float32)
    g = h * dis[:, None]
    out_ref[0] = g[:, :128]
    out_ref[1] = g[:, 128:]


def _tc_c_body(s_ref, deg_ref, b_ref, out_ref):
    dis = _dis_from(deg_ref)
    sfull = jnp.concatenate([s_ref[0], s_ref[1]], axis=1)
    out_ref[...] = sfull * dis[:, None] + b_ref[0]


_tc_a0 = pl.pallas_call(
    _tc_a0_body,
    grid=(N // R,),
    in_specs=[
        pl.BlockSpec((R, 256), lambda i: (i, 0)),
        pl.BlockSpec((256, 128), lambda i: (0, 0)),
    ],
    out_specs=pl.BlockSpec((R, 128), lambda i: (i, 0)),
    out_shape=jax.ShapeDtypeStruct((N, 128), jnp.float32),
)

_tc_a1 = pl.pallas_call(
    _tc_a1_body,
    grid=(N // R,),
    in_specs=[
        pl.BlockSpec((R, 128), lambda i: (i, 0)),
        pl.BlockSpec((2, R, 16), lambda i: (0, i, 0)),
    ],
    out_specs=pl.BlockSpec((R, 128), lambda i: (i, 0)),
    out_shape=jax.ShapeDtypeStruct((N, 128), jnp.float32),
)

_tc_b = pl.pallas_call(
    _tc_b_body,
    grid=(N // R,),
    in_specs=[
        pl.BlockSpec((2, R, 128), lambda i: (0, i, 0)),
        pl.BlockSpec((R, 128), lambda i: (i, 0)),
        pl.BlockSpec((2, R, 16), lambda i: (0, i, 0)),
        pl.BlockSpec((1, 128), lambda i: (0, 0)),
        pl.BlockSpec((128, 256), lambda i: (0, 0)),
    ],
    out_specs=pl.BlockSpec((2, R, 128), lambda i: (0, i, 0)),
    out_shape=jax.ShapeDtypeStruct((2, N, 128), jnp.float32),
)

_tc_c = pl.pallas_call(
    _tc_c_body,
    grid=(N // R,),
    in_specs=[
        pl.BlockSpec((2, R, 128), lambda i: (0, i, 0)),
        pl.BlockSpec((2, R, 16), lambda i: (0, i, 0)),
        pl.BlockSpec((1, 256), lambda i: (0, 0)),
    ],
    out_specs=pl.BlockSpec((R, 256), lambda i: (i, 0)),
    out_shape=jax.ShapeDtypeStruct((N, 256), jnp.float32),
)


@jax.jit
def kernel(x, edge_index, W1, b1, W2, b2):
    row = edge_index[0]
    col = edge_index[1]
    rowcat = jnp.concatenate([row, row + N])
    zeros16 = jnp.zeros((NTM, 16), jnp.float32)
    ones16 = jnp.ones((CD, 16), jnp.float32)

    deg2 = _deg_kernel(col, zeros16, ones16).reshape(2, N, 16)
    h1 = _tc_a0(x, W1)

    row2d = row.reshape(-1, 50)
    col2d = col.reshape(-1, 50)
    g1 = _tc_a1(h1, deg2)
    s1 = _agg_l1(g1, row2d, col2d).reshape(2, N, 128)

    g2 = _tc_b(s1, g1, deg2, b1.reshape(1, 128), W2).reshape(2 * N, 128)
    s2 = _agg_l2(g2, rowcat.reshape(-1, 50), col2d).reshape(2, N, 128)

    return _tc_c(s2, deg2, b2.reshape(1, 256))


# speedup vs baseline: 22.6024x; 1.0142x over previous
import functools

import jax
import jax.numpy as jnp
from jax import lax
from jax.experimental import pallas as pl
from jax.experimental.pallas import tpu as pltpu
from jax.experimental.pallas import tpu_sc as plsc

N = 10000
E = 160000
NC = 2
NS = 16
NTM = 624
REM = N - NTM * NS
ET = E // NS
ED = E // (NC * NS)
CD = 1000

R = 1000

_MESH = plsc.VectorSubcoreMesh(
    core_axis_name="c", subcore_axis_name="s", num_cores=NC, num_subcores=NS)
_SC_PARAMS = pltpu.CompilerParams(use_tc_tiling_on_sc=False,
                                  needs_layout_passes=False)


@functools.partial(
    pl.kernel,
    out_type=jax.ShapeDtypeStruct((NC * N,), jnp.float32),
    mesh=_MESH,
    compiler_params=_SC_PARAMS,
    scratch_types=[
        pltpu.VMEM_SHARED((N, 16), jnp.float32),
        pltpu.VMEM((NTM, 16), jnp.float32),
        pltpu.VMEM((CD, 16), jnp.float32),
        pltpu.VMEM((CD,), jnp.int32),
        pltpu.VMEM((NTM,), jnp.float32),
        pltpu.SemaphoreType.DMA,
    ],
)
def _deg_kernel(col_hbm, zeros_hbm, ones_hbm, out_hbm, acc, zed_v, ones_v,
                icol_v, lane0_v, sem):
    c = lax.axis_index("c")
    s = lax.axis_index("s")
    pltpu.sync_copy(zeros_hbm.at[pl.ds(0, NTM), :], zed_v)
    pltpu.sync_copy(ones_hbm.at[pl.ds(0, CD), :], ones_v)
    pltpu.sync_copy(zed_v, acc.at[pl.ds(s * NTM, NTM), :])

    @pl.when(s == 0)
    def _():
        pltpu.sync_copy(zed_v.at[pl.ds(0, REM), :],
                        acc.at[pl.ds(NTM * NS, REM), :])

    plsc.subcore_barrier()
    wid = s * NC + c
    e_base = wid * ED

    def body(i, _):
        pltpu.sync_copy(col_hbm.at[pl.ds(e_base + i * CD, CD)], icol_v)
        pltpu.sync_copy(ones_v, acc.at[icol_v], add=True)
        return 0

    lax.fori_loop(0, ED // CD, body, 0)
    plsc.subcore_barrier()
    pltpu.sync_copy(acc.at[pl.ds(s * NTM, NTM), :], zed_v)
    rows16 = lax.iota(jnp.int32, 16)
    col0 = jnp.zeros((16,), jnp.int32)

    def extract(j, _):
        lane0_v[pl.ds(j * 16, 16)] = plsc.load_gather(
            zed_v, [rows16 + j * 16, col0])
        return 0

    lax.fori_loop(0, NTM // 16, extract, 0)
    pltpu.sync_copy(lane0_v, out_hbm.at[pl.ds(c * N + s * NTM, NTM)])

    @pl.when(s == 0)
    def _():
        pltpu.sync_copy(acc.at[pl.ds(NTM * NS, REM), :],
                        zed_v.at[pl.ds(0, REM), :])
        lane0_v[pl.ds(0, 16)] = plsc.load_gather(zed_v, [rows16, col0])
        pltpu.sync_copy(lane0_v.at[pl.ds(0, REM)],
                        out_hbm.at[pl.ds(c * N + NTM * NS, REM)])


RING = 4


def _make_agg(dh, ca, esplit):
    et = E // NC // NS if esplit else ET
    nch = et // ca
    nq = nch // RING

    def idxrow(idx2d, chunk):
        return idx2d.at[chunk]

    @functools.partial(
        pl.kernel,
        out_type=jax.ShapeDtypeStruct((NC * N, dh), jnp.float32),
        mesh=_MESH,
        compiler_params=_SC_PARAMS,
        scratch_types=[
            pltpu.VMEM_SHARED((N, dh), jnp.float32),
            pltpu.VMEM((nch, ca), jnp.int32),
            pltpu.VMEM((nch, ca), jnp.int32),
            [pltpu.VMEM((ca, dh), jnp.float32)] * RING,
            [pltpu.SemaphoreType.DMA] * RING,
            [pltpu.SemaphoreType.DMA] * RING,
        ],
    )
    def agg(g_hbm, row2d_hbm, col2d_hbm, out_hbm, acc, irow2d, icol2d,
            bufs, semg, sems):
        c = lax.axis_index("c")
        s = lax.axis_index("s")
        ginit = 0 if esplit else c * N
        pltpu.sync_copy(g_hbm.at[pl.ds(ginit + s * NTM, NTM), :],
                        acc.at[pl.ds(s * NTM, NTM), :])

        @pl.when(s == 0)
        def _():
            pltpu.sync_copy(g_hbm.at[pl.ds(ginit + NTM * NS, REM), :],
                            acc.at[pl.ds(NTM * NS, REM), :])

        if esplit:
            rbase = (c * (E // NC) + s * et) // ca
            cbase = rbase
        else:
            rbase = (c * E + s * et) // ca
            cbase = s * et // ca
        pltpu.sync_copy(row2d_hbm.at[pl.ds(rbase, nch), :], irow2d)
        pltpu.sync_copy(col2d_hbm.at[pl.ds(cbase, nch), :], icol2d)
        plsc.subcore_barrier()

        def gat(chunk, b):
            return pltpu.make_async_copy(
                g_hbm.at[idxrow(irow2d, chunk)], bufs[b], semg[b])

        def sca(chunk, b):
            return pltpu.make_async_copy(
                bufs[b], acc.at[idxrow(icol2d, chunk)], sems[b])

        for b in range(RING):
            gat(b, b).start()

        def body(k, _):
            base = k * RING
            for b in range(RING):
                chunk = base + b
                gat(chunk, b).wait()
                pltpu.async_copy(
                    bufs[b], acc.at[idxrow(icol2d, chunk)], sems[b],
                    add=True)

                @pl.when(k < nq - 1)
                def _():
                    sca(chunk, b).wait()
                    gat(chunk + RING, b).start()

            return 0

        lax.fori_loop(0, nq, body, 0)
        for b in range(RING):
            sca(nch - RING + b, b).wait()
        plsc.subcore_barrier()
        pltpu.sync_copy(acc.at[pl.ds(s * NTM, NTM), :],
                        out_hbm.at[pl.ds(c * N + s * NTM, NTM), :])

        @pl.when(s == 0)
        def _():
            pltpu.sync_copy(acc.at[pl.ds(NTM * NS, REM), :],
                            out_hbm.at[pl.ds(c * N + NTM * NS, REM), :])

    return agg


_agg_l1 = _make_agg(128, 50, esplit=True)
_agg_l2 = _make_agg(128, 50, esplit=False)


def _dis_from(dt_ref):
    dt = dt_ref[...]
    return lax.rsqrt(dt[:, 0] + dt[:, 1] + 1.0)


def _tc_a0_body(x_ref, w_ref, out_ref):
    out_ref[...] = jnp.dot(x_ref[...], w_ref[...],
                           preferred_element_type=jnp.float32)


def _tc_a1_body(h_ref, dt_ref, out_ref):
    out_ref[...] = h_ref[...] * _dis_from(dt_ref)[:, None]


def _tc_b_body(s_ref, g1_ref, dt_ref, b_ref, w_ref, out_ref):
    dis = _dis_from(dt_ref)
    sfull = s_ref[0] + s_ref[1] - g1_ref[...]
    x2 = jnp.maximum(sfull * dis[:, None] + b_ref[0], 0.0)
    h = jnp.dot(x2, w_ref[...], preferred_element_type=jnp.float32)
    g = h * dis[:, None]
    out_ref[0] = g[:, :128]
    out_ref[1] = g[:, 128:]


def _tc_c_body(s_ref, dt_ref, b_ref, out_ref):
    dis = _dis_from(dt_ref)
    sfull = jnp.concatenate([s_ref[0], s_ref[1]], axis=1)
    out_ref[...] = sfull * dis[:, None] + b_ref[0]


def _deg_specs():
    return [pl.BlockSpec((R, 2), lambda i: (i, 0))]


_tc_a0 = pl.pallas_call(
    _tc_a0_body,
    grid=(N // R,),
    in_specs=[
        pl.BlockSpec((R, 256), lambda i: (i, 0)),
        pl.BlockSpec((256, 128), lambda i: (0, 0)),
    ],
    out_specs=pl.BlockSpec((R, 128), lambda i: (i, 0)),
    out_shape=jax.ShapeDtypeStruct((N, 128), jnp.float32),
)

_tc_a1 = pl.pallas_call(
    _tc_a1_body,
    grid=(N // R,),
    in_specs=[
        pl.BlockSpec((R, 128), lambda i: (i, 0)),
        *_deg_specs(),
    ],
    out_specs=pl.BlockSpec((R, 128), lambda i: (i, 0)),
    out_shape=jax.ShapeDtypeStruct((N, 128), jnp.float32),
)

_tc_b = pl.pallas_call(
    _tc_b_body,
    grid=(N // R,),
    in_specs=[
        pl.BlockSpec((2, R, 128), lambda i: (0, i, 0)),
        pl.BlockSpec((R, 128), lambda i: (i, 0)),
        *_deg_specs(),
        pl.BlockSpec((1, 128), lambda i: (0, 0)),
        pl.BlockSpec((128, 256), lambda i: (0, 0)),
    ],
    out_specs=pl.BlockSpec((2, R, 128), lambda i: (0, i, 0)),
    out_shape=jax.ShapeDtypeStruct((2, N, 128), jnp.float32),
)

_tc_c = pl.pallas_call(
    _tc_c_body,
    grid=(N // R,),
    in_specs=[
        pl.BlockSpec((2, R, 128), lambda i: (0, i, 0)),
        *_deg_specs(),
        pl.BlockSpec((1, 256), lambda i: (0, 0)),
    ],
    out_specs=pl.BlockSpec((R, 256), lambda i: (i, 0)),
    out_shape=jax.ShapeDtypeStruct((N, 256), jnp.float32),
)


@jax.jit
def kernel(x, edge_index, W1, b1, W2, b2):
    row = edge_index[0]
    col = edge_index[1]
    rowcat = jnp.concatenate([row, row + N])
    zeros16 = jnp.zeros((NTM, 16), jnp.float32)
    ones16 = jnp.ones((CD, 16), jnp.float32)

    degt = _deg_kernel(col, zeros16, ones16).reshape(2, N).T
    h1 = _tc_a0(x, W1)

    row2d = row.reshape(-1, 50)
    col2d = col.reshape(-1, 50)
    g1 = _tc_a1(h1, degt)
    s1 = _agg_l1(g1, row2d, col2d).reshape(2, N, 128)

    g2 = _tc_b(s1, g1, degt, b1.reshape(1, 128), W2)
    s2 = _agg_l2(g2.reshape(2 * N, 128), rowcat.reshape(-1, 50), col2d)

    return _tc_c(s2.reshape(2, N, 128), degt, b2.reshape(1, 256))
